# baseline (speedup 1.0000x reference)
import functools

import jax
import jax.numpy as jnp
import numpy as np
from jax import lax
from jax.experimental import pallas as pl
from jax.experimental.pallas import tpu as pltpu
from jax.experimental.pallas import tpu_sc as plsc

N = 10000
E = 320000
D_NODE = 128
D_EDGE = 16
H1 = 128
H2 = 128
OUT = 1

NC = 2
NS = 16
NW = NC * NS
CH = 128
K_CHUNKS = 80
E_PAD = NW * CH * K_CHUNKS
NPAD = 10112
RPT = NPAD // NS

_f32 = jnp.float32


def _sc_mesh():
    return plsc.VectorSubcoreMesh(core_axis_name="c", subcore_axis_name="s")


def _zero_acc_rows(srcbuf, acc, r0, nrows):
    done = 0
    while done < nrows:
        step = min(CH, nrows - done)
        pltpu.sync_copy(srcbuf.at[pl.ds(0, step)],
                        acc.at[pl.ds(r0 + done, step)])
        done += step


def _fill_128(buf, vec16):
    def fill(i, _):
        for j in range(8):
            buf[i, pl.ds(j * 16, 16)] = vec16
        return 0
    lax.fori_loop(0, CH, fill, 0)


@functools.partial(
    pl.kernel,
    out_type=jax.ShapeDtypeStruct((NC, NPAD, 128), _f32),
    mesh=_sc_mesh(),
    scratch_types=[
        pltpu.VMEM((CH,), jnp.int32),
        pltpu.VMEM((CH,), jnp.int32),
        pltpu.VMEM((CH // 8, 128), _f32),
        pltpu.VMEM((CH, 128), _f32),
        pltpu.VMEM((CH, 128), _f32),
        pltpu.VMEM_SHARED((NPAD, 128), _f32),
    ],
)
def _sc_edscatter(eat_h, src_h, dst_h, out_o, didxv, sidxv, eatp, rows, ones32, acc):
    c = lax.axis_index("c")
    s = lax.axis_index("s")
    wid = s * NC + c

    zv = jnp.zeros((16,), _f32)
    e0 = jnp.where(lax.iota(jnp.int32, 16) == 0, 1.0, 0.0).astype(_f32)

    _fill_128(rows, zv)
    _fill_128(ones32, zv)
    r0 = s * RPT
    _zero_acc_rows(rows, acc, r0, RPT)

    def fill_cols(i, _):
        rows[i, pl.ds(16, 16)] = e0
        ones32[i, pl.ds(32, 16)] = e0
        return 0

    lax.fori_loop(0, CH, fill_cols, 0)
    plsc.subcore_barrier()

    def step(k, _):
        basep = (wid * K_CHUNKS + k) * (CH // 8)
        pltpu.sync_copy(dst_h.at[wid, k], didxv)
        pltpu.sync_copy(src_h.at[wid, k], sidxv)
        pltpu.sync_copy(eat_h.at[pl.ds(basep, CH // 8)], eatp)
        for i in range(CH):
            rows[i, pl.ds(0, 16)] = eatp[i // 8, pl.ds((i % 8) * 16, 16)]
        pltpu.sync_copy(rows, acc.at[didxv], add=True)
        pltpu.sync_copy(ones32, acc.at[sidxv], add=True)
        return 0

    lax.fori_loop(0, K_CHUNKS, step, 0)
    plsc.subcore_barrier()
    pltpu.sync_copy(acc.at[pl.ds(r0, RPT)], out_o.at[c, pl.ds(r0, RPT)])


@functools.partial(
    pl.kernel,
    out_type=jax.ShapeDtypeStruct((NC, NPAD, H1), _f32),
    mesh=_sc_mesh(),
    scratch_types=[
        pltpu.VMEM((CH,), jnp.int32),
        pltpu.VMEM((CH,), jnp.int32),
        pltpu.VMEM((CH, H1), _f32),
        pltpu.VMEM_SHARED((NPAD, H1), _f32),
        pltpu.SemaphoreType.DMA,
        pltpu.SemaphoreType.DMA,
    ],
)
def _sc_segsum(tab_h, src_h, dst_h, out_o, sidx, didx, rows, acc, sem, sem2):
    c = lax.axis_index("c")
    s = lax.axis_index("s")
    wid = s * NC + c

    _fill_128(rows, jnp.zeros((16,), _f32))
    r0 = s * RPT
    _zero_acc_rows(rows, acc, r0, RPT)
    plsc.subcore_barrier()

    def step(k, _):
        base = (wid * K_CHUNKS + k) * CH
        pltpu.sync_copy(src_h.at[pl.ds(base, CH)], sidx)
        pltpu.sync_copy(dst_h.at[pl.ds(base, CH)], didx)
        half = CH // 2
        d1 = pltpu.async_copy(tab_h.at[sidx.at[pl.ds(0, half)]],
                              rows.at[pl.ds(0, half)], sem)
        d2 = pltpu.async_copy(tab_h.at[sidx.at[pl.ds(half, half)]],
                              rows.at[pl.ds(half, half)], sem2)
        d1.wait()
        d2.wait()
        pltpu.sync_copy(rows, acc.at[didx], add=True)
        return 0

    lax.fori_loop(0, K_CHUNKS, step, 0)
    plsc.subcore_barrier()
    pltpu.sync_copy(acc.at[pl.ds(r0, RPT)], out_o.at[c, pl.ds(r0, RPT)])


_BT = 632
_GT = NPAD // _BT


def _k1_body(x_ref, w_ref, o_ref):
    o_ref[...] = jnp.dot(x_ref[...], w_ref[...], preferred_element_type=_f32)


def _tc_k1(x_pad, W1n):
    return pl.pallas_call(
        _k1_body,
        grid=(_GT,),
        in_specs=[
            pl.BlockSpec((_BT, D_NODE), lambda i: (i, 0)),
            pl.BlockSpec((D_NODE, H1), lambda i: (0, 0)),
        ],
        out_specs=pl.BlockSpec((_BT, H1), lambda i: (i, 0)),
        out_shape=jax.ShapeDtypeStruct((NPAD, H1), _f32),
    )(x_pad, W1n)


def _k2_body(agg_ref, ed_ref, w1e_ref, b1_ref, w2_ref, o_ref):
    a = agg_ref[0] + agg_ref[1]
    se = ed_ref[0, :, :D_EDGE] + ed_ref[1, :, :D_EDGE]
    h = jnp.maximum(
        a + jnp.dot(se, w1e_ref[...], preferred_element_type=_f32) + b1_ref[...],
        0.0)
    od = ed_ref[0, :, 32:33] + ed_ref[1, :, 32:33]
    sn = lax.rsqrt(jnp.maximum(od, 1.0))
    o_ref[...] = jnp.dot(h, w2_ref[...], preferred_element_type=_f32) * sn


def _tc_k2(agg, ed, W1e, b1r, W2):
    return pl.pallas_call(
        _k2_body,
        grid=(_GT,),
        in_specs=[
            pl.BlockSpec((NC, _BT, H1), lambda i: (0, i, 0)),
            pl.BlockSpec((NC, _BT, 128), lambda i: (0, i, 0)),
            pl.BlockSpec((D_EDGE, H1), lambda i: (0, 0)),
            pl.BlockSpec((1, H1), lambda i: (0, 0)),
            pl.BlockSpec((H1, H2), lambda i: (0, 0)),
        ],
        out_specs=pl.BlockSpec((_BT, H2), lambda i: (i, 0)),
        out_shape=jax.ShapeDtypeStruct((NPAD, H2), _f32),
    )(agg, ed, W1e, b1r, W2)


_BM = 1000
_GM = N // _BM


def _k3_body(a2_ref, ed_ref, b2_ref, wp_ref, bp_ref, o_ref, accs):
    i = pl.program_id(0)

    @pl.when(i == 0)
    def _():
        accs[...] = jnp.zeros_like(accs)

    a = a2_ref[0] + a2_ref[1]
    din = ed_ref[0, :, D_EDGE:D_EDGE + 1] + ed_ref[1, :, D_EDGE:D_EDGE + 1]
    dn = lax.rsqrt(jnp.maximum(din, 1.0))
    h2 = jnp.maximum(a * dn + b2_ref[...], 0.0)
    accs[...] += jnp.sum(h2, axis=0, keepdims=True)

    @pl.when(i == _GM - 1)
    def _():
        hg = accs[...] * np.float32(1.0 / N)
        o_ref[...] = jnp.sum(hg * wp_ref[...], axis=1, keepdims=True) + bp_ref[...]


def _tc_k3(a2, ed, b2r, wpr, bpr):
    return pl.pallas_call(
        _k3_body,
        grid=(_GM,),
        in_specs=[
            pl.BlockSpec((NC, _BM, H2), lambda i: (0, i, 0)),
            pl.BlockSpec((NC, _BM, 128), lambda i: (0, i, 0)),
            pl.BlockSpec((1, H2), lambda i: (0, 0)),
            pl.BlockSpec((1, H2), lambda i: (0, 0)),
            pl.BlockSpec((1, 1), lambda i: (0, 0)),
        ],
        out_specs=pl.BlockSpec((1, 1), lambda i: (0, 0)),
        out_shape=jax.ShapeDtypeStruct((1, 1), _f32),
        scratch_shapes=[pltpu.VMEM((1, H2), _f32)],
    )(a2, ed, b2r, wpr, bpr)


@jax.jit
def kernel(node_features, edge_index, edge_features, W1n, W1e, b1, W2, b2, Wp, bp):
    src = edge_index[0]
    dst = edge_index[1]
    npad_rows = NPAD - N
    epad = E_PAD - E
    srcp = jnp.concatenate([src, jnp.full((epad,), N, jnp.int32)])
    dstp = jnp.concatenate([dst, jnp.full((epad,), N, jnp.int32)])
    srcp3 = srcp.reshape(NW, K_CHUNKS, CH)
    dstp3 = dstp.reshape(NW, K_CHUNKS, CH)
    x_pad = jnp.concatenate([node_features, jnp.zeros((npad_rows, D_NODE), _f32)])
    eatp = jnp.concatenate([edge_features,
                            jnp.zeros((epad, D_EDGE), _f32)]).reshape(
                                E_PAD // 8, 128)

    xw = _tc_k1(x_pad, W1n)
    ed = _sc_edscatter(eatp, srcp3, dstp3)
    agg = _sc_segsum(xw, srcp, dstp)
    v = _tc_k2(agg, ed, W1e, b1.reshape(1, H1), W2)
    a2 = _sc_segsum(v, srcp, dstp)
    out = _tc_k3(a2, ed, b2.reshape(1, H2), Wp.reshape(1, H2), bp.reshape(1, 1))
    return out.reshape(OUT)


# speedup vs baseline: 1.0935x; 1.0935x over previous
import functools

import jax
import jax.numpy as jnp
import numpy as np
from jax import lax
from jax.experimental import pallas as pl
from jax.experimental.pallas import tpu as pltpu
from jax.experimental.pallas import tpu_sc as plsc

N = 10000
E = 320000
D_NODE = 128
D_EDGE = 16
H1 = 128
H2 = 128
OUT = 1

NC = 2
NS = 16
NW = NC * NS
CH = 128
K_CHUNKS = 79
E_PAD = NW * CH * K_CHUNKS
NPAD = 10112
RPT = NPAD // NS

_f32 = jnp.float32


def _sc_mesh():
    return plsc.VectorSubcoreMesh(core_axis_name="c", subcore_axis_name="s")


def _zero_acc_rows(srcbuf, acc, r0, nrows):
    done = 0
    while done < nrows:
        step = min(CH, nrows - done)
        pltpu.sync_copy(srcbuf.at[pl.ds(0, step)],
                        acc.at[pl.ds(r0 + done, step)])
        done += step


def _fill_128(buf, vec16):
    def fill(i, _):
        for j in range(8):
            buf[i, pl.ds(j * 16, 16)] = vec16
        return 0
    lax.fori_loop(0, CH, fill, 0)


@functools.partial(
    pl.kernel,
    out_type=jax.ShapeDtypeStruct((NC, NPAD, 128), _f32),
    mesh=_sc_mesh(),
    scratch_types=[
        pltpu.VMEM((CH,), jnp.int32),
        pltpu.VMEM((CH, 128), _f32),
        pltpu.VMEM_SHARED((NPAD, 128), _f32),
    ],
)
def _sc_rowscatter(rows_h, idx_h, out_o, idxv, rows, acc):
    c = lax.axis_index("c")
    s = lax.axis_index("s")
    wid = s * NC + c

    _fill_128(rows, jnp.zeros((16,), _f32))
    r0 = s * RPT
    _zero_acc_rows(rows, acc, r0, RPT)
    plsc.subcore_barrier()

    def step(k, _):
        base = (wid * K_CHUNKS + k) * CH
        pltpu.sync_copy(idx_h.at[pl.ds(base, CH)], idxv)
        pltpu.sync_copy(rows_h.at[pl.ds(base, CH)], rows)
        pltpu.sync_copy(rows, acc.at[idxv], add=True)
        return 0

    lax.fori_loop(0, K_CHUNKS, step, 0)
    plsc.subcore_barrier()
    pltpu.sync_copy(acc.at[pl.ds(r0, RPT)], out_o.at[c, pl.ds(r0, RPT)])


@functools.partial(
    pl.kernel,
    out_type=jax.ShapeDtypeStruct((NC, NPAD, 128), _f32),
    mesh=_sc_mesh(),
    scratch_types=[
        pltpu.VMEM((CH,), jnp.int32),
        pltpu.VMEM((CH, 128), _f32),
        pltpu.VMEM((CH, 128), _f32),
        pltpu.VMEM_SHARED((NPAD, 128), _f32),
    ],
)
def _sc_histogram(idx_h, out_o, idxv, zrows, ones, acc):
    c = lax.axis_index("c")
    s = lax.axis_index("s")
    wid = s * NC + c

    _fill_128(zrows, jnp.zeros((16,), _f32))
    _fill_128(ones, jnp.ones((16,), _f32))
    r0 = s * RPT
    _zero_acc_rows(zrows, acc, r0, RPT)
    plsc.subcore_barrier()

    def step(k, _):
        base = (wid * K_CHUNKS + k) * CH
        pltpu.sync_copy(idx_h.at[pl.ds(base, CH)], idxv)
        pltpu.sync_copy(ones, acc.at[idxv], add=True)
        return 0

    lax.fori_loop(0, K_CHUNKS, step, 0)
    plsc.subcore_barrier()
    pltpu.sync_copy(acc.at[pl.ds(r0, RPT)], out_o.at[c, pl.ds(r0, RPT)])


@functools.partial(
    pl.kernel,
    out_type=jax.ShapeDtypeStruct((NC, NPAD, H1), _f32),
    mesh=_sc_mesh(),
    scratch_types=[
        pltpu.VMEM((CH,), jnp.int32),
        pltpu.VMEM((CH,), jnp.int32),
        pltpu.VMEM((CH, H1), _f32),
        pltpu.VMEM_SHARED((NPAD, H1), _f32),
        pltpu.SemaphoreType.DMA,
    ],
)
def _sc_segsum(tab_h, src_h, dst_h, out_o, sidx, didx, rows, acc, sem):
    c = lax.axis_index("c")
    s = lax.axis_index("s")
    wid = s * NC + c

    _fill_128(rows, jnp.zeros((16,), _f32))
    r0 = s * RPT
    _zero_acc_rows(rows, acc, r0, RPT)
    plsc.subcore_barrier()

    def step(k, _):
        base = (wid * K_CHUNKS + k) * CH
        pltpu.sync_copy(src_h.at[pl.ds(base, CH)], sidx)
        pltpu.sync_copy(dst_h.at[pl.ds(base, CH)], didx)
        pltpu.async_copy(tab_h.at[sidx], rows, sem).wait()
        pltpu.sync_copy(rows, acc.at[didx], add=True)
        return 0

    lax.fori_loop(0, K_CHUNKS, step, 0)
    plsc.subcore_barrier()
    pltpu.sync_copy(acc.at[pl.ds(r0, RPT)], out_o.at[c, pl.ds(r0, RPT)])


_BT = 632
_GT = NPAD // _BT


def _k1_body(x_ref, w_ref, o_ref):
    o_ref[...] = jnp.dot(x_ref[...], w_ref[...], preferred_element_type=_f32)


def _tc_k1(x_pad, W1n):
    return pl.pallas_call(
        _k1_body,
        grid=(_GT,),
        in_specs=[
            pl.BlockSpec((_BT, D_NODE), lambda i: (i, 0)),
            pl.BlockSpec((D_NODE, H1), lambda i: (0, 0)),
        ],
        out_specs=pl.BlockSpec((_BT, H1), lambda i: (i, 0)),
        out_shape=jax.ShapeDtypeStruct((NPAD, H1), _f32),
    )(x_pad, W1n)


def _k2_body(agg_ref, ed_ref, do_ref, w1e_ref, b1_ref, w2_ref, o_ref):
    a = agg_ref[0] + agg_ref[1]
    se = ed_ref[0, :, :D_EDGE] + ed_ref[1, :, :D_EDGE]
    h = jnp.maximum(
        a + jnp.dot(se, w1e_ref[...], preferred_element_type=_f32) + b1_ref[...],
        0.0)
    od = do_ref[0, :, :1] + do_ref[1, :, :1]
    sn = lax.rsqrt(jnp.maximum(od, 1.0))
    o_ref[...] = jnp.dot(h, w2_ref[...], preferred_element_type=_f32) * sn


def _tc_k2(agg, ed, do, W1e, b1r, W2):
    return pl.pallas_call(
        _k2_body,
        grid=(_GT,),
        in_specs=[
            pl.BlockSpec((NC, _BT, H1), lambda i: (0, i, 0)),
            pl.BlockSpec((NC, _BT, 128), lambda i: (0, i, 0)),
            pl.BlockSpec((NC, _BT, 128), lambda i: (0, i, 0)),
            pl.BlockSpec((D_EDGE, H1), lambda i: (0, 0)),
            pl.BlockSpec((1, H1), lambda i: (0, 0)),
            pl.BlockSpec((H1, H2), lambda i: (0, 0)),
        ],
        out_specs=pl.BlockSpec((_BT, H2), lambda i: (i, 0)),
        out_shape=jax.ShapeDtypeStruct((NPAD, H2), _f32),
    )(agg, ed, do, W1e, b1r, W2)


_BM = 1000
_GM = N // _BM


def _k3_body(a2_ref, ed_ref, b2_ref, wp_ref, bp_ref, o_ref, accs):
    i = pl.program_id(0)

    @pl.when(i == 0)
    def _():
        accs[...] = jnp.zeros_like(accs)

    a = a2_ref[0] + a2_ref[1]
    din = ed_ref[0, :, D_EDGE:D_EDGE + 1] + ed_ref[1, :, D_EDGE:D_EDGE + 1]
    dn = lax.rsqrt(jnp.maximum(din, 1.0))
    h2 = jnp.maximum(a * dn + b2_ref[...], 0.0)
    accs[...] += jnp.sum(h2, axis=0, keepdims=True)

    @pl.when(i == _GM - 1)
    def _():
        hg = accs[...] * np.float32(1.0 / N)
        o_ref[...] = jnp.sum(hg * wp_ref[...], axis=1, keepdims=True) + bp_ref[...]


def _tc_k3(a2, ed, b2r, wpr, bpr):
    return pl.pallas_call(
        _k3_body,
        grid=(_GM,),
        in_specs=[
            pl.BlockSpec((NC, _BM, H2), lambda i: (0, i, 0)),
            pl.BlockSpec((NC, _BM, 128), lambda i: (0, i, 0)),
            pl.BlockSpec((1, H2), lambda i: (0, 0)),
            pl.BlockSpec((1, H2), lambda i: (0, 0)),
            pl.BlockSpec((1, 1), lambda i: (0, 0)),
        ],
        out_specs=pl.BlockSpec((1, 1), lambda i: (0, 0)),
        out_shape=jax.ShapeDtypeStruct((1, 1), _f32),
        scratch_shapes=[pltpu.VMEM((1, H2), _f32)],
    )(a2, ed, b2r, wpr, bpr)


@jax.jit
def kernel(node_features, edge_index, edge_features, W1n, W1e, b1, W2, b2, Wp, bp):
    src = edge_index[0]
    dst = edge_index[1]
    npad_rows = NPAD - N
    epad = E_PAD - E
    srcp = jnp.concatenate([src, jnp.full((epad,), N, jnp.int32)])
    dstp = jnp.concatenate([dst, jnp.full((epad,), N, jnp.int32)])
    x_pad = jnp.concatenate([node_features, jnp.zeros((npad_rows, D_NODE), _f32)])
    eatw = jnp.zeros((E_PAD, 128), _f32)
    eatw = eatw.at[:E, :D_EDGE].set(edge_features)
    eatw = eatw.at[:E, D_EDGE].set(1.0)

    xw = _tc_k1(x_pad, W1n)
    ed = _sc_rowscatter(eatw, dstp)
    do = _sc_histogram(srcp)
    agg = _sc_segsum(xw, srcp, dstp)
    v = _tc_k2(agg, ed, do, W1e, b1.reshape(1, H1), W2)
    a2 = _sc_segsum(v, srcp, dstp)
    out = _tc_k3(a2, ed, b2.reshape(1, H2), Wp.reshape(1, H2), bp.reshape(1, 1))
    return out.reshape(OUT)
